# baseline (speedup 1.0000x reference)
import functools

import jax
import jax.numpy as jnp
from jax import lax
from jax.experimental import pallas as pl
from jax.experimental.pallas import tpu as pltpu
from jax.experimental.pallas import tpu_sc as plsc

_B = 16384
_C = 1000
_BLK = 2048
_NC, _NS, _L = 2, 16, 16
_NW = _NC * _NS
_R = 16
_S = 0

_LN2 = 0.6931471805599453
_P = (1.4720650105289716e-06, 0.9998476974962408, -0.4973732161580116,
      0.3157473167582232, -0.19035433673352936, 0.08269123711180565,
      -0.017414077524380872)


def _softplus16(y):
    a = jnp.exp(y) + 1.0
    i = plsc.bitcast(a, jnp.int32)
    e = (i >> 23) - 127
    m = plsc.bitcast((i & 0x007FFFFF) | 0x3F800000, jnp.float32)
    f = m - 1.0
    p = jnp.float32(_P[6])
    for k in (5, 4, 3, 2, 1, 0):
        p = p * f + jnp.float32(_P[k])
    return e.astype(jnp.float32) * jnp.float32(_LN2) + p


_rpw = max(_S, _NW * _R) // _NW
_n_chunks = _rpw // _R


def _sc_body(logits_hbm, tgt_hbm, cost_hbm, out_hbm,
             idx_v, cost_v, logi_v, acc_v, gsem, lsem):
    wid = lax.axis_index("s") * _NC + lax.axis_index("c")
    base = wid * _rpw
    pltpu.sync_copy(tgt_hbm.at[pl.ds(base, _rpw)], idx_v)

    def start(c, buf):
        pltpu.async_copy(cost_hbm.at[idx_v.at[pl.ds(c * _R, _R)]],
                         cost_v.at[buf], gsem)
        pltpu.async_copy(logits_hbm.at[pl.ds(base + c * _R, _R), :],
                         logi_v.at[buf], lsem)

    def wait(buf):
        pltpu.make_async_copy(cost_hbm.at[pl.ds(0, _R)], cost_v.at[buf], gsem).wait()
        pltpu.make_async_copy(logits_hbm.at[pl.ds(0, _R), :], logi_v.at[buf], lsem).wait()

    start(0, 0)
    lane = lax.iota(jnp.int32, _L)

    def chunk_body(c, acc):
        buf = lax.rem(c, 2)

        @pl.when(c + 1 < _n_chunks)
        def _():
            start(c + 1, 1 - buf)

        wait(buf)

        def row_body(r, acc):
            def col_body(j, acc):
                y = (logi_v[buf, r, pl.ds(j * _L, _L)]
                     - cost_v[buf, r, pl.ds(j * _L, _L)])
                return acc + _softplus16(y)
            acc = lax.fori_loop(0, 62, col_body, acc, unroll=4)
            y = (logi_v[buf, r, pl.ds(984, _L)]
                 - cost_v[buf, r, pl.ds(984, _L)])
            sp = _softplus16(y)
            return acc + jnp.where(lane >= 8, sp, 0.0)

        acc = lax.fori_loop(0, _R, row_body, acc)

        def corr_body(g, acc):
            t16 = idx_v[pl.ds(c * _R + g * _L, _L)]
            rows = lane + g * _L
            lg = plsc.load_gather(logi_v.at[buf], [rows, t16])
            cg = plsc.load_gather(cost_v.at[buf], [rows, t16])
            return acc - (lg - cg)
        acc = lax.fori_loop(0, _R // _L, corr_body, acc)
        return acc

    acc = lax.fori_loop(0, _n_chunks, chunk_body, jnp.zeros((_L,), jnp.float32))
    acc_v[...] = acc
    pltpu.sync_copy(acc_v, out_hbm.at[pl.ds(wid * _L, _L)])


_sc_partial = functools.partial(
    pl.kernel,
    mesh=plsc.VectorSubcoreMesh(core_axis_name="c", subcore_axis_name="s"),
    out_type=jax.ShapeDtypeStruct((_NW * _L,), jnp.float32),
    compiler_params=pltpu.CompilerParams(needs_layout_passes=False,
                                         use_tc_tiling_on_sc=False),
    scratch_types=[
        pltpu.VMEM((_rpw,), jnp.int32),
        pltpu.VMEM((2, _R, _C), jnp.float32),
        pltpu.VMEM((2, _R, _C), jnp.float32),
        pltpu.VMEM((_L,), jnp.float32),
        pltpu.SemaphoreType.DMA,
        pltpu.SemaphoreType.DMA,
    ],
)(_sc_body)


_NSPLIT = 16
_SUB = _BLK // _NSPLIT


def _tc_body(*refs):
    tgt_refs = refs[:_NSPLIT]
    logit_refs = refs[_NSPLIT:2 * _NSPLIT]
    cost_ref = refs[2 * _NSPLIT]
    out_ref = refs[2 * _NSPLIT + 1]

    lane = jax.lax.broadcasted_iota(jnp.int32, (_SUB, _C), 1)
    acc = jnp.zeros((_C,), jnp.float32)
    for h in range(_NSPLIT):
        t = tgt_refs[h][...]
        eq = lane == t
        onehot = jnp.where(eq, jnp.float32(1.0), jnp.float32(0.0)).astype(jnp.bfloat16)
        ct = jnp.dot(onehot, cost_ref[...], preferred_element_type=jnp.float32)
        y = logit_refs[h][...] - ct
        sp = jnp.log1p(jnp.exp(y)) - jnp.where(eq, y, jnp.float32(0.0))
        acc = acc + jnp.sum(sp, axis=0)
    part = jnp.sum(acc, keepdims=True).reshape(1, 1)

    @pl.when(pl.program_id(0) == 0)
    def _init():
        out_ref[...] = jnp.zeros_like(out_ref)

    out_ref[...] += part


def kernel(logits, targets, cost_matrix):
    t32 = targets.astype(jnp.int32)
    if _S:
        sc_parts = _sc_partial(logits[:_S], t32[:_S], cost_matrix)

    t2 = t32.reshape(_B, 1)
    cbf = cost_matrix.astype(jnp.bfloat16)
    off = _S // _SUB
    tgt_specs = [
        pl.BlockSpec((_SUB, 1), functools.partial(lambda h, i: (off + _NSPLIT * i + h, 0), h))
        for h in range(_NSPLIT)
    ]
    logit_specs = [
        pl.BlockSpec((_SUB, _C), functools.partial(lambda h, i: (off + _NSPLIT * i + h, 0), h))
        for h in range(_NSPLIT)
    ]
    tc_total = pl.pallas_call(
        _tc_body,
        grid=((_B - _S) // _BLK,),
        in_specs=tgt_specs + logit_specs + [pl.BlockSpec((_C, _C), lambda i: (0, 0))],
        out_specs=pl.BlockSpec((1, 1), lambda i: (0, 0)),
        out_shape=jax.ShapeDtypeStruct((1, 1), jnp.float32),
    )(*([t2] * _NSPLIT), *([logits] * _NSPLIT), cbf)
    total = tc_total[0, 0]
    if _S:
        total = total + jnp.sum(sc_parts)
    return (total / (_B * _C)).astype(jnp.float32)


# speedup vs baseline: 1.0099x; 1.0099x over previous
import functools

import jax
import jax.numpy as jnp
from jax import lax
from jax.experimental import pallas as pl
from jax.experimental.pallas import tpu as pltpu
from jax.experimental.pallas import tpu_sc as plsc

_B = 16384
_C = 1000
_BLK = 2048
_NC, _NS, _L = 2, 16, 16
_NW = _NC * _NS
_R = 16
_S = 0

_LN2 = 0.6931471805599453
_P = (1.4720650105289716e-06, 0.9998476974962408, -0.4973732161580116,
      0.3157473167582232, -0.19035433673352936, 0.08269123711180565,
      -0.017414077524380872)


def _softplus16(y):
    a = jnp.exp(y) + 1.0
    i = plsc.bitcast(a, jnp.int32)
    e = (i >> 23) - 127
    m = plsc.bitcast((i & 0x007FFFFF) | 0x3F800000, jnp.float32)
    f = m - 1.0
    p = jnp.float32(_P[6])
    for k in (5, 4, 3, 2, 1, 0):
        p = p * f + jnp.float32(_P[k])
    return e.astype(jnp.float32) * jnp.float32(_LN2) + p


_rpw = max(_S, _NW * _R) // _NW
_n_chunks = _rpw // _R


def _sc_body(logits_hbm, tgt_hbm, cost_hbm, out_hbm,
             idx_v, cost_v, logi_v, acc_v, gsem, lsem):
    wid = lax.axis_index("s") * _NC + lax.axis_index("c")
    base = wid * _rpw
    pltpu.sync_copy(tgt_hbm.at[pl.ds(base, _rpw)], idx_v)

    def start(c, buf):
        pltpu.async_copy(cost_hbm.at[idx_v.at[pl.ds(c * _R, _R)]],
                         cost_v.at[buf], gsem)
        pltpu.async_copy(logits_hbm.at[pl.ds(base + c * _R, _R), :],
                         logi_v.at[buf], lsem)

    def wait(buf):
        pltpu.make_async_copy(cost_hbm.at[pl.ds(0, _R)], cost_v.at[buf], gsem).wait()
        pltpu.make_async_copy(logits_hbm.at[pl.ds(0, _R), :], logi_v.at[buf], lsem).wait()

    start(0, 0)
    lane = lax.iota(jnp.int32, _L)

    def chunk_body(c, acc):
        buf = lax.rem(c, 2)

        @pl.when(c + 1 < _n_chunks)
        def _():
            start(c + 1, 1 - buf)

        wait(buf)

        def row_body(r, acc):
            def col_body(j, acc):
                y = (logi_v[buf, r, pl.ds(j * _L, _L)]
                     - cost_v[buf, r, pl.ds(j * _L, _L)])
                return acc + _softplus16(y)
            acc = lax.fori_loop(0, 62, col_body, acc, unroll=4)
            y = (logi_v[buf, r, pl.ds(984, _L)]
                 - cost_v[buf, r, pl.ds(984, _L)])
            sp = _softplus16(y)
            return acc + jnp.where(lane >= 8, sp, 0.0)

        acc = lax.fori_loop(0, _R, row_body, acc)

        def corr_body(g, acc):
            t16 = idx_v[pl.ds(c * _R + g * _L, _L)]
            rows = lane + g * _L
            lg = plsc.load_gather(logi_v.at[buf], [rows, t16])
            cg = plsc.load_gather(cost_v.at[buf], [rows, t16])
            return acc - (lg - cg)
        acc = lax.fori_loop(0, _R // _L, corr_body, acc)
        return acc

    acc = lax.fori_loop(0, _n_chunks, chunk_body, jnp.zeros((_L,), jnp.float32))
    acc_v[...] = acc
    pltpu.sync_copy(acc_v, out_hbm.at[pl.ds(wid * _L, _L)])


_sc_partial = functools.partial(
    pl.kernel,
    mesh=plsc.VectorSubcoreMesh(core_axis_name="c", subcore_axis_name="s"),
    out_type=jax.ShapeDtypeStruct((_NW * _L,), jnp.float32),
    compiler_params=pltpu.CompilerParams(needs_layout_passes=False,
                                         use_tc_tiling_on_sc=False),
    scratch_types=[
        pltpu.VMEM((_rpw,), jnp.int32),
        pltpu.VMEM((2, _R, _C), jnp.float32),
        pltpu.VMEM((2, _R, _C), jnp.float32),
        pltpu.VMEM((_L,), jnp.float32),
        pltpu.SemaphoreType.DMA,
        pltpu.SemaphoreType.DMA,
    ],
)(_sc_body)


_NSPLIT = 8
_SUB = _BLK // _NSPLIT


def _tc_body(*refs):
    tgt_refs = refs[:_NSPLIT]
    logit_refs = refs[_NSPLIT:2 * _NSPLIT]
    cost_ref = refs[2 * _NSPLIT]
    out_ref = refs[2 * _NSPLIT + 1]

    lane = jax.lax.broadcasted_iota(jnp.int32, (_SUB, _C), 1)
    acc = jnp.zeros((_C,), jnp.float32)
    for h in range(_NSPLIT):
        t = tgt_refs[h][...]
        eq = lane == t
        onehot = jnp.where(eq, jnp.float32(1.0), jnp.float32(0.0)).astype(jnp.bfloat16)
        ct = jnp.dot(onehot, cost_ref[...], preferred_element_type=jnp.float32)
        y = logit_refs[h][...] - ct
        sp = jnp.log1p(jnp.exp(y)) - jnp.where(eq, y, jnp.float32(0.0))
        acc = acc + jnp.sum(sp, axis=0)
    part = jnp.sum(acc, keepdims=True).reshape(1, 1)

    @pl.when(pl.program_id(0) == 0)
    def _init():
        out_ref[...] = jnp.zeros_like(out_ref)

    out_ref[...] += part


def kernel(logits, targets, cost_matrix):
    t32 = targets.astype(jnp.int32)
    if _S:
        sc_parts = _sc_partial(logits[:_S], t32[:_S], cost_matrix)

    t2 = t32.reshape(_B, 1)
    cbf = cost_matrix.astype(jnp.bfloat16)
    off = _S // _SUB
    tgt_specs = [
        pl.BlockSpec((_SUB, 1), functools.partial(lambda h, i: (off + _NSPLIT * i + h, 0), h))
        for h in range(_NSPLIT)
    ]
    logit_specs = [
        pl.BlockSpec((_SUB, _C), functools.partial(lambda h, i: (off + _NSPLIT * i + h, 0), h))
        for h in range(_NSPLIT)
    ]
    tc_total = pl.pallas_call(
        _tc_body,
        grid=((_B - _S) // _BLK,),
        in_specs=tgt_specs + logit_specs + [pl.BlockSpec((_C, _C), lambda i: (0, 0))],
        out_specs=pl.BlockSpec((1, 1), lambda i: (0, 0)),
        out_shape=jax.ShapeDtypeStruct((1, 1), jnp.float32),
    )(*([t2] * _NSPLIT), *([logits] * _NSPLIT), cbf)
    total = tc_total[0, 0]
    if _S:
        total = total + jnp.sum(sc_parts)
    return (total / (_B * _C)).astype(jnp.float32)


# speedup vs baseline: 1.8112x; 1.7934x over previous
import functools

import jax
import jax.numpy as jnp
from jax import lax
from jax.experimental import pallas as pl
from jax.experimental.pallas import tpu as pltpu
from jax.experimental.pallas import tpu_sc as plsc

_B = 16384
_C = 1000
_BLK = 2048
_NC, _NS, _L = 2, 16, 16
_NW = _NC * _NS
_R = 16
_S = 0

_LN2 = 0.6931471805599453
_P = (1.4720650105289716e-06, 0.9998476974962408, -0.4973732161580116,
      0.3157473167582232, -0.19035433673352936, 0.08269123711180565,
      -0.017414077524380872)


def _softplus16(y):
    a = jnp.exp(y) + 1.0
    i = plsc.bitcast(a, jnp.int32)
    e = (i >> 23) - 127
    m = plsc.bitcast((i & 0x007FFFFF) | 0x3F800000, jnp.float32)
    f = m - 1.0
    p = jnp.float32(_P[6])
    for k in (5, 4, 3, 2, 1, 0):
        p = p * f + jnp.float32(_P[k])
    return e.astype(jnp.float32) * jnp.float32(_LN2) + p


_rpw = max(_S, _NW * _R) // _NW
_n_chunks = _rpw // _R


def _sc_body(logits_hbm, tgt_hbm, cost_hbm, out_hbm,
             idx_v, cost_v, logi_v, acc_v, gsem, lsem):
    wid = lax.axis_index("s") * _NC + lax.axis_index("c")
    base = wid * _rpw
    pltpu.sync_copy(tgt_hbm.at[pl.ds(base, _rpw)], idx_v)

    def start(c, buf):
        pltpu.async_copy(cost_hbm.at[idx_v.at[pl.ds(c * _R, _R)]],
                         cost_v.at[buf], gsem)
        pltpu.async_copy(logits_hbm.at[pl.ds(base + c * _R, _R), :],
                         logi_v.at[buf], lsem)

    def wait(buf):
        pltpu.make_async_copy(cost_hbm.at[pl.ds(0, _R)], cost_v.at[buf], gsem).wait()
        pltpu.make_async_copy(logits_hbm.at[pl.ds(0, _R), :], logi_v.at[buf], lsem).wait()

    start(0, 0)
    lane = lax.iota(jnp.int32, _L)

    def chunk_body(c, acc):
        buf = lax.rem(c, 2)

        @pl.when(c + 1 < _n_chunks)
        def _():
            start(c + 1, 1 - buf)

        wait(buf)

        def row_body(r, acc):
            def col_body(j, acc):
                y = (logi_v[buf, r, pl.ds(j * _L, _L)]
                     - cost_v[buf, r, pl.ds(j * _L, _L)])
                return acc + _softplus16(y)
            acc = lax.fori_loop(0, 62, col_body, acc, unroll=4)
            y = (logi_v[buf, r, pl.ds(984, _L)]
                 - cost_v[buf, r, pl.ds(984, _L)])
            sp = _softplus16(y)
            return acc + jnp.where(lane >= 8, sp, 0.0)

        acc = lax.fori_loop(0, _R, row_body, acc)

        def corr_body(g, acc):
            t16 = idx_v[pl.ds(c * _R + g * _L, _L)]
            rows = lane + g * _L
            lg = plsc.load_gather(logi_v.at[buf], [rows, t16])
            cg = plsc.load_gather(cost_v.at[buf], [rows, t16])
            return acc - (lg - cg)
        acc = lax.fori_loop(0, _R // _L, corr_body, acc)
        return acc

    acc = lax.fori_loop(0, _n_chunks, chunk_body, jnp.zeros((_L,), jnp.float32))
    acc_v[...] = acc
    pltpu.sync_copy(acc_v, out_hbm.at[pl.ds(wid * _L, _L)])


_sc_partial = functools.partial(
    pl.kernel,
    mesh=plsc.VectorSubcoreMesh(core_axis_name="c", subcore_axis_name="s"),
    out_type=jax.ShapeDtypeStruct((_NW * _L,), jnp.float32),
    compiler_params=pltpu.CompilerParams(needs_layout_passes=False,
                                         use_tc_tiling_on_sc=False),
    scratch_types=[
        pltpu.VMEM((_rpw,), jnp.int32),
        pltpu.VMEM((2, _R, _C), jnp.float32),
        pltpu.VMEM((2, _R, _C), jnp.float32),
        pltpu.VMEM((_L,), jnp.float32),
        pltpu.SemaphoreType.DMA,
        pltpu.SemaphoreType.DMA,
    ],
)(_sc_body)


def _tc_body(tgt_ref, logits_ref, cost_ref, out_ref):
    t = tgt_ref[...]
    cls = jax.lax.broadcasted_iota(jnp.int32, (_C, _BLK), 0)
    eq = cls == t
    onehot = jnp.where(eq, jnp.float32(1.0), jnp.float32(0.0)).astype(jnp.bfloat16)
    ct = jnp.dot(cost_ref[...], onehot, preferred_element_type=jnp.float32)
    y = logits_ref[...] - ct
    sp = jnp.log1p(jnp.exp(y)) - jnp.where(eq, y, jnp.float32(0.0))
    part = jnp.sum(jnp.sum(sp, axis=1), keepdims=True).reshape(1, 1)

    @pl.when(pl.program_id(0) == 0)
    def _init():
        out_ref[...] = jnp.zeros_like(out_ref)

    out_ref[...] += part


def kernel(logits, targets, cost_matrix):
    t32 = targets.astype(jnp.int32)
    if _S:
        sc_parts = _sc_partial(logits[:_S], t32[:_S], cost_matrix)

    logits_t = jnp.transpose(logits)
    t2 = t32.reshape(1, _B)
    cbf_t = jnp.transpose(cost_matrix).astype(jnp.bfloat16)
    off = _S // _BLK
    tc_total = pl.pallas_call(
        _tc_body,
        grid=((_B - _S) // _BLK,),
        in_specs=[
            pl.BlockSpec((1, _BLK), lambda i: (0, off + i)),
            pl.BlockSpec((_C, _BLK), lambda i: (0, off + i)),
            pl.BlockSpec((_C, _C), lambda i: (0, 0)),
        ],
        out_specs=pl.BlockSpec((1, 1), lambda i: (0, 0)),
        out_shape=jax.ShapeDtypeStruct((1, 1), jnp.float32),
    )(t2, logits_t, cbf_t)
    total = tc_total[0, 0]
    if _S:
        total = total + jnp.sum(sc_parts)
    return (total / (_B * _C)).astype(jnp.float32)


# speedup vs baseline: 2.0813x; 1.1492x over previous
import functools

import jax
import jax.numpy as jnp
from jax import lax
from jax.experimental import pallas as pl
from jax.experimental.pallas import tpu as pltpu
from jax.experimental.pallas import tpu_sc as plsc

_B = 16384
_C = 1000
_BLK = 2048
_NC, _NS, _L = 2, 16, 16
_NW = _NC * _NS
_R = 16
_S = 0

_LN2 = 0.6931471805599453
_P = (1.4720650105289716e-06, 0.9998476974962408, -0.4973732161580116,
      0.3157473167582232, -0.19035433673352936, 0.08269123711180565,
      -0.017414077524380872)


def _softplus16(y):
    a = jnp.exp(y) + 1.0
    i = plsc.bitcast(a, jnp.int32)
    e = (i >> 23) - 127
    m = plsc.bitcast((i & 0x007FFFFF) | 0x3F800000, jnp.float32)
    f = m - 1.0
    p = jnp.float32(_P[6])
    for k in (5, 4, 3, 2, 1, 0):
        p = p * f + jnp.float32(_P[k])
    return e.astype(jnp.float32) * jnp.float32(_LN2) + p


_rpw = max(_S, _NW * _R) // _NW
_n_chunks = _rpw // _R


def _sc_body(logits_hbm, tgt_hbm, cost_hbm, out_hbm,
             idx_v, cost_v, logi_v, acc_v, gsem, lsem):
    wid = lax.axis_index("s") * _NC + lax.axis_index("c")
    base = wid * _rpw
    pltpu.sync_copy(tgt_hbm.at[pl.ds(base, _rpw)], idx_v)

    def start(c, buf):
        pltpu.async_copy(cost_hbm.at[idx_v.at[pl.ds(c * _R, _R)]],
                         cost_v.at[buf], gsem)
        pltpu.async_copy(logits_hbm.at[pl.ds(base + c * _R, _R), :],
                         logi_v.at[buf], lsem)

    def wait(buf):
        pltpu.make_async_copy(cost_hbm.at[pl.ds(0, _R)], cost_v.at[buf], gsem).wait()
        pltpu.make_async_copy(logits_hbm.at[pl.ds(0, _R), :], logi_v.at[buf], lsem).wait()

    start(0, 0)
    lane = lax.iota(jnp.int32, _L)

    def chunk_body(c, acc):
        buf = lax.rem(c, 2)

        @pl.when(c + 1 < _n_chunks)
        def _():
            start(c + 1, 1 - buf)

        wait(buf)

        def row_body(r, acc):
            def col_body(j, acc):
                y = (logi_v[buf, r, pl.ds(j * _L, _L)]
                     - cost_v[buf, r, pl.ds(j * _L, _L)])
                return acc + _softplus16(y)
            acc = lax.fori_loop(0, 62, col_body, acc, unroll=4)
            y = (logi_v[buf, r, pl.ds(984, _L)]
                 - cost_v[buf, r, pl.ds(984, _L)])
            sp = _softplus16(y)
            return acc + jnp.where(lane >= 8, sp, 0.0)

        acc = lax.fori_loop(0, _R, row_body, acc)

        def corr_body(g, acc):
            t16 = idx_v[pl.ds(c * _R + g * _L, _L)]
            rows = lane + g * _L
            lg = plsc.load_gather(logi_v.at[buf], [rows, t16])
            cg = plsc.load_gather(cost_v.at[buf], [rows, t16])
            return acc - (lg - cg)
        acc = lax.fori_loop(0, _R // _L, corr_body, acc)
        return acc

    acc = lax.fori_loop(0, _n_chunks, chunk_body, jnp.zeros((_L,), jnp.float32))
    acc_v[...] = acc
    pltpu.sync_copy(acc_v, out_hbm.at[pl.ds(wid * _L, _L)])


_sc_partial = functools.partial(
    pl.kernel,
    mesh=plsc.VectorSubcoreMesh(core_axis_name="c", subcore_axis_name="s"),
    out_type=jax.ShapeDtypeStruct((_NW * _L,), jnp.float32),
    compiler_params=pltpu.CompilerParams(needs_layout_passes=False,
                                         use_tc_tiling_on_sc=False),
    scratch_types=[
        pltpu.VMEM((_rpw,), jnp.int32),
        pltpu.VMEM((2, _R, _C), jnp.float32),
        pltpu.VMEM((2, _R, _C), jnp.float32),
        pltpu.VMEM((_L,), jnp.float32),
        pltpu.SemaphoreType.DMA,
        pltpu.SemaphoreType.DMA,
    ],
)(_sc_body)


def _tc_body(tgt_ref, logits_ref, cost_ref, out_ref):
    t = tgt_ref[...]
    cls = jax.lax.broadcasted_iota(jnp.int32, (_C, _BLK), 0)
    eq = cls == t
    onehot = jnp.where(eq, jnp.float32(1.0), jnp.float32(0.0)).astype(jnp.bfloat16)
    ct = jnp.dot(cost_ref[...], onehot, preferred_element_type=jnp.float32)
    y = logits_ref[...] - ct
    sp = jnp.log(1.0 + jnp.exp(y)) - jnp.where(eq, y, jnp.float32(0.0))
    part = jnp.sum(jnp.sum(sp, axis=0), keepdims=True).reshape(1, 1)

    @pl.when(pl.program_id(0) == 0)
    def _init():
        out_ref[...] = jnp.zeros_like(out_ref)

    out_ref[...] += part


def kernel(logits, targets, cost_matrix):
    t32 = targets.astype(jnp.int32)
    if _S:
        sc_parts = _sc_partial(logits[:_S], t32[:_S], cost_matrix)

    logits_t = jnp.transpose(logits)
    t2 = t32.reshape(1, _B)
    cbf_t = jnp.transpose(cost_matrix).astype(jnp.bfloat16)
    off = _S // _BLK
    tc_total = pl.pallas_call(
        _tc_body,
        grid=((_B - _S) // _BLK,),
        in_specs=[
            pl.BlockSpec((1, _BLK), lambda i: (0, off + i)),
            pl.BlockSpec((_C, _BLK), lambda i: (0, off + i)),
            pl.BlockSpec((_C, _C), lambda i: (0, 0)),
        ],
        out_specs=pl.BlockSpec((1, 1), lambda i: (0, 0)),
        out_shape=jax.ShapeDtypeStruct((1, 1), jnp.float32),
    )(t2, logits_t, cbf_t)
    total = tc_total[0, 0]
    if _S:
        total = total + jnp.sum(sc_parts)
    return (total / (_B * _C)).astype(jnp.float32)


# speedup vs baseline: 2.0840x; 1.0013x over previous
import functools

import jax
import jax.numpy as jnp
from jax import lax
from jax.experimental import pallas as pl
from jax.experimental.pallas import tpu as pltpu
from jax.experimental.pallas import tpu_sc as plsc

_B = 16384
_C = 1000
_BLK = 2048
_NC, _NS, _L = 2, 16, 16
_NW = _NC * _NS
_R = 16
_S = 0

_LN2 = 0.6931471805599453
_P = (1.4720650105289716e-06, 0.9998476974962408, -0.4973732161580116,
      0.3157473167582232, -0.19035433673352936, 0.08269123711180565,
      -0.017414077524380872)


def _softplus16(y):
    a = jnp.exp(y) + 1.0
    i = plsc.bitcast(a, jnp.int32)
    e = (i >> 23) - 127
    m = plsc.bitcast((i & 0x007FFFFF) | 0x3F800000, jnp.float32)
    f = m - 1.0
    p = jnp.float32(_P[6])
    for k in (5, 4, 3, 2, 1, 0):
        p = p * f + jnp.float32(_P[k])
    return e.astype(jnp.float32) * jnp.float32(_LN2) + p


_rpw = max(_S, _NW * _R) // _NW
_n_chunks = _rpw // _R


def _sc_body(logits_hbm, tgt_hbm, cost_hbm, out_hbm,
             idx_v, cost_v, logi_v, acc_v, gsem, lsem):
    wid = lax.axis_index("s") * _NC + lax.axis_index("c")
    base = wid * _rpw
    pltpu.sync_copy(tgt_hbm.at[pl.ds(base, _rpw)], idx_v)

    def start(c, buf):
        pltpu.async_copy(cost_hbm.at[idx_v.at[pl.ds(c * _R, _R)]],
                         cost_v.at[buf], gsem)
        pltpu.async_copy(logits_hbm.at[pl.ds(base + c * _R, _R), :],
                         logi_v.at[buf], lsem)

    def wait(buf):
        pltpu.make_async_copy(cost_hbm.at[pl.ds(0, _R)], cost_v.at[buf], gsem).wait()
        pltpu.make_async_copy(logits_hbm.at[pl.ds(0, _R), :], logi_v.at[buf], lsem).wait()

    start(0, 0)
    lane = lax.iota(jnp.int32, _L)

    def chunk_body(c, acc):
        buf = lax.rem(c, 2)

        @pl.when(c + 1 < _n_chunks)
        def _():
            start(c + 1, 1 - buf)

        wait(buf)

        def row_body(r, acc):
            def col_body(j, acc):
                y = (logi_v[buf, r, pl.ds(j * _L, _L)]
                     - cost_v[buf, r, pl.ds(j * _L, _L)])
                return acc + _softplus16(y)
            acc = lax.fori_loop(0, 62, col_body, acc, unroll=4)
            y = (logi_v[buf, r, pl.ds(984, _L)]
                 - cost_v[buf, r, pl.ds(984, _L)])
            sp = _softplus16(y)
            return acc + jnp.where(lane >= 8, sp, 0.0)

        acc = lax.fori_loop(0, _R, row_body, acc)

        def corr_body(g, acc):
            t16 = idx_v[pl.ds(c * _R + g * _L, _L)]
            rows = lane + g * _L
            lg = plsc.load_gather(logi_v.at[buf], [rows, t16])
            cg = plsc.load_gather(cost_v.at[buf], [rows, t16])
            return acc - (lg - cg)
        acc = lax.fori_loop(0, _R // _L, corr_body, acc)
        return acc

    acc = lax.fori_loop(0, _n_chunks, chunk_body, jnp.zeros((_L,), jnp.float32))
    acc_v[...] = acc
    pltpu.sync_copy(acc_v, out_hbm.at[pl.ds(wid * _L, _L)])


_sc_partial = functools.partial(
    pl.kernel,
    mesh=plsc.VectorSubcoreMesh(core_axis_name="c", subcore_axis_name="s"),
    out_type=jax.ShapeDtypeStruct((_NW * _L,), jnp.float32),
    compiler_params=pltpu.CompilerParams(needs_layout_passes=False,
                                         use_tc_tiling_on_sc=False),
    scratch_types=[
        pltpu.VMEM((_rpw,), jnp.int32),
        pltpu.VMEM((2, _R, _C), jnp.float32),
        pltpu.VMEM((2, _R, _C), jnp.float32),
        pltpu.VMEM((_L,), jnp.float32),
        pltpu.SemaphoreType.DMA,
        pltpu.SemaphoreType.DMA,
    ],
)(_sc_body)


_TSPLIT = 2
_TSUB = _BLK // _TSPLIT


def _tc_body(*refs):
    tgt_refs = refs[:_TSPLIT]
    logit_refs = refs[_TSPLIT:2 * _TSPLIT]
    cost_ref = refs[2 * _TSPLIT]
    out_ref = refs[2 * _TSPLIT + 1]

    cls = jax.lax.broadcasted_iota(jnp.int32, (_C, _TSUB), 0)
    acc = jnp.zeros((_TSUB,), jnp.float32)
    for h in range(_TSPLIT):
        t = tgt_refs[h][...]
        eq = cls == t
        onehot = jnp.where(eq, jnp.float32(1.0), jnp.float32(0.0)).astype(jnp.bfloat16)
        ct = jnp.dot(cost_ref[...], onehot, preferred_element_type=jnp.float32)
        y = logit_refs[h][...] - ct
        sp = jnp.log(1.0 + jnp.exp(y)) - jnp.where(eq, y, jnp.float32(0.0))
        acc = acc + jnp.sum(sp, axis=0)
    part = jnp.sum(acc, keepdims=True).reshape(1, 1)

    @pl.when(pl.program_id(0) == 0)
    def _init():
        out_ref[...] = jnp.zeros_like(out_ref)

    out_ref[...] += part


def kernel(logits, targets, cost_matrix):
    t32 = targets.astype(jnp.int32)
    if _S:
        sc_parts = _sc_partial(logits[:_S], t32[:_S], cost_matrix)

    logits_t = jnp.transpose(logits)
    t2 = t32.reshape(1, _B)
    cbf_t = jnp.transpose(cost_matrix).astype(jnp.bfloat16)
    off = _S // _TSUB
    tgt_specs = [
        pl.BlockSpec((1, _TSUB), functools.partial(lambda h, i: (0, off + _TSPLIT * i + h), h))
        for h in range(_TSPLIT)
    ]
    logit_specs = [
        pl.BlockSpec((_C, _TSUB), functools.partial(lambda h, i: (0, off + _TSPLIT * i + h), h))
        for h in range(_TSPLIT)
    ]
    tc_total = pl.pallas_call(
        _tc_body,
        grid=((_B - _S) // _BLK,),
        in_specs=tgt_specs + logit_specs + [pl.BlockSpec((_C, _C), lambda i: (0, 0))],
        out_specs=pl.BlockSpec((1, 1), lambda i: (0, 0)),
        out_shape=jax.ShapeDtypeStruct((1, 1), jnp.float32),
    )(*([t2] * _TSPLIT), *([logits_t] * _TSPLIT), cbf_t)
    total = tc_total[0, 0]
    if _S:
        total = total + jnp.sum(sc_parts)
    return (total / (_B * _C)).astype(jnp.float32)
